# baseline (speedup 1.0000x reference)
import functools

import jax
import jax.numpy as jnp
import numpy as np
from jax import lax
from jax.experimental import pallas as pl
from jax.experimental.pallas import tpu as pltpu
from jax.experimental.pallas import tpu_sc as plsc

NUM_LEVELS = 16
NUM_FEATS = 2
LOG2_T = 19
T = 2 ** LOG2_T
BASE_RES = 16.0
DESIRED_RES = 2048.0
N_POINTS = 262144

P1 = np.int32(np.int64(2654435761) - 2 ** 32)
P2 = np.int32(805459861)
HASH_MASK = T - 1

NC = 2
NS = 16
LANES = 16
NW = NC * NS
PTS_PER_WORKER = N_POINTS // NW
CHUNK = 1024
N_CHUNKS = PTS_PER_WORKER // CHUNK
GROUPS = CHUNK // LANES
ENC_DIM = NUM_LEVELS * NUM_FEATS

_growth = np.exp((np.log(DESIRED_RES) - np.log(BASE_RES)) / (NUM_LEVELS - 1))
RES_F32 = np.float32(BASE_RES * _growth ** np.arange(NUM_LEVELS))


def _sc_encode_body(x_hbm, y_hbm, z_hbm, tbl_hbm, res_hbm, enc_hbm,
                    xs, ys, zs, idx1d, w2d, rows, enc_buf, res_v, sem):
    wid = lax.axis_index("s") * NC + lax.axis_index("c")
    pltpu.sync_copy(res_hbm, res_v)
    lane = lax.iota(jnp.int32, LANES)
    zeros16 = jnp.zeros((LANES,), jnp.int32)
    ones16 = jnp.ones((LANES,), jnp.int32)

    def chunk_body(ci, _):
        base = wid * PTS_PER_WORKER + ci * CHUNK
        pltpu.sync_copy(x_hbm.at[pl.ds(base, CHUNK)], xs)
        pltpu.sync_copy(y_hbm.at[pl.ds(base, CHUNK)], ys)
        pltpu.sync_copy(z_hbm.at[pl.ds(base, CHUNK)], zs)

        def level_body(l, _):
            res = plsc.load_gather(res_v, [jnp.full((LANES,), l, jnp.int32)])
            lofs = l * T

            def grp1(j, _):
                x = xs[pl.ds(j * LANES, LANES)]
                y = ys[pl.ds(j * LANES, LANES)]
                z = zs[pl.ds(j * LANES, LANES)]
                sx = x * res
                sy = y * res
                sz = z * res
                fx = sx.astype(jnp.int32)
                fy = sy.astype(jnp.int32)
                fz = sz.astype(jnp.int32)
                frx = sx - fx.astype(jnp.float32)
                fry = sy - fy.astype(jnp.float32)
                frz = sz - fz.astype(jnp.float32)
                a = (fx, fx + 1)
                b = (fy * P1, fy * P1 + P1)
                c = (fz * P2, fz * P2 + P2)
                wx = (1.0 - frx, frx)
                wy = (1.0 - fry, fry)
                wz = (1.0 - frz, frz)
                for corner in range(8):
                    cx, cy, cz = corner & 1, (corner >> 1) & 1, (corner >> 2) & 1
                    h = ((a[cx] ^ b[cy] ^ c[cz]) & HASH_MASK) + lofs
                    idx1d[pl.ds(j * 8 * LANES + corner * LANES, LANES)] = h
                    w2d[j, pl.ds(corner * LANES, LANES)] = (wx[cx] * wy[cy]) * wz[cz]
                return 0

            lax.fori_loop(0, GROUPS, grp1, 0)
            pltpu.async_copy(tbl_hbm.at[idx1d], rows, sem).wait()

            def grp2(j, _):
                f0 = jnp.zeros((LANES,), jnp.float32)
                f1 = jnp.zeros((LANES,), jnp.float32)
                for corner in range(8):
                    wv = w2d[j, pl.ds(corner * LANES, LANES)]
                    li = lane + (j * 8 * LANES + corner * LANES)
                    r0 = plsc.load_gather(rows, [li, zeros16])
                    r1 = plsc.load_gather(rows, [li, ones16])
                    f0 = f0 + wv * r0
                    f1 = f1 + wv * r1
                prow = j * LANES + lane
                plsc.store_scatter(enc_buf, [prow, jnp.full((LANES,), 2 * l, jnp.int32)], f0)
                plsc.store_scatter(enc_buf, [prow, jnp.full((LANES,), 2 * l + 1, jnp.int32)], f1)
                return 0

            lax.fori_loop(0, GROUPS, grp2, 0)
            return 0

        lax.fori_loop(0, NUM_LEVELS, level_body, 0)
        pltpu.sync_copy(enc_buf, enc_hbm.at[pl.ds(base, CHUNK)])
        return 0

    lax.fori_loop(0, N_CHUNKS, chunk_body, 0)


@jax.jit
def _sc_encode(x, y, z, tbl, res_arr):
    mesh = plsc.VectorSubcoreMesh(core_axis_name="c", subcore_axis_name="s")
    f = pl.kernel(
        _sc_encode_body,
        out_type=jax.ShapeDtypeStruct((N_POINTS, ENC_DIM), jnp.float32),
        mesh=mesh,
        compiler_params=pltpu.CompilerParams(
            needs_layout_passes=False, use_tc_tiling_on_sc=False),
        scratch_types=[
            pltpu.VMEM((CHUNK,), jnp.float32),
            pltpu.VMEM((CHUNK,), jnp.float32),
            pltpu.VMEM((CHUNK,), jnp.float32),
            pltpu.VMEM((GROUPS * 8 * LANES,), jnp.int32),
            pltpu.VMEM((GROUPS, 8 * LANES), jnp.float32),
            pltpu.VMEM((GROUPS * 8 * LANES, NUM_FEATS), jnp.float32),
            pltpu.VMEM((CHUNK, ENC_DIM), jnp.float32),
            pltpu.VMEM((NUM_LEVELS,), jnp.float32),
            pltpu.SemaphoreType.DMA,
        ],
    )
    return f(x, y, z, tbl, res_arr)


def _mlp_body(enc_ref, w1_ref, b1_ref, w2_ref, b2_ref, out_ref):
    h = jnp.dot(enc_ref[...], w1_ref[...], preferred_element_type=jnp.float32)
    h = jnp.maximum(h + b1_ref[...], 0.0)
    out_ref[...] = jnp.dot(h, w2_ref[...], preferred_element_type=jnp.float32) + b2_ref[...]


@jax.jit
def _mlp(enc, W1, b1, W2, b2):
    BN = 4096
    grid = (N_POINTS // BN,)
    return pl.pallas_call(
        _mlp_body,
        grid=grid,
        in_specs=[
            pl.BlockSpec((BN, ENC_DIM), lambda i: (i, 0)),
            pl.BlockSpec((ENC_DIM, 64), lambda i: (0, 0)),
            pl.BlockSpec((1, 64), lambda i: (0, 0)),
            pl.BlockSpec((64, NUM_FEATS), lambda i: (0, 0)),
            pl.BlockSpec((1, NUM_FEATS), lambda i: (0, 0)),
        ],
        out_specs=pl.BlockSpec((BN, NUM_FEATS), lambda i: (i, 0)),
        out_shape=jax.ShapeDtypeStruct((N_POINTS, NUM_FEATS), jnp.float32),
    )(enc, W1, b1, W2, b2)


def kernel(coords, tables, W1, b1, W2, b2):
    x = coords[:, 0]
    y = coords[:, 1]
    z = coords[:, 2]
    tbl = tables.reshape(NUM_LEVELS * T, NUM_FEATS)
    res_arr = jnp.asarray(RES_F32)
    enc = _sc_encode(x, y, z, tbl, res_arr)
    return _mlp(enc, W1, b1.reshape(1, 64), W2, b2.reshape(1, NUM_FEATS))


# speedup vs baseline: 1.1370x; 1.1370x over previous
import functools

import jax
import jax.numpy as jnp
import numpy as np
from jax import lax
from jax.experimental import pallas as pl
from jax.experimental.pallas import tpu as pltpu
from jax.experimental.pallas import tpu_sc as plsc

NUM_LEVELS = 16
NUM_FEATS = 2
LOG2_T = 19
T = 2 ** LOG2_T
BASE_RES = 16.0
DESIRED_RES = 2048.0
N_POINTS = 262144

P1 = np.int32(np.int64(2654435761) - 2 ** 32)
P2 = np.int32(805459861)
HASH_MASK = T - 1

NC = 2
NS = 16
LANES = 16
NW = NC * NS
PTS_PER_WORKER = N_POINTS // NW
CHUNK = 1024
N_CHUNKS = PTS_PER_WORKER // CHUNK
GROUPS = CHUNK // LANES
ENC_DIM = NUM_LEVELS * NUM_FEATS

_growth = np.exp((np.log(DESIRED_RES) - np.log(BASE_RES)) / (NUM_LEVELS - 1))
RES_F32 = np.float32(BASE_RES * _growth ** np.arange(NUM_LEVELS))


def _sc_encode_body(x_hbm, y_hbm, z_hbm, tbl_hbm, res_hbm, enc_hbm,
                    xs, ys, zs, idxA, idxB, w2d, rows0, rows1, enc_buf,
                    res_v, semA, semB):
    wid = lax.axis_index("s") * NC + lax.axis_index("c")
    pltpu.sync_copy(res_hbm, res_v)
    lane = lax.iota(jnp.int32, LANES)

    def chunk_body(ci, _):
        base = wid * PTS_PER_WORKER + ci * CHUNK
        pltpu.sync_copy(x_hbm.at[pl.ds(base, CHUNK)], xs)
        pltpu.sync_copy(y_hbm.at[pl.ds(base, CHUNK)], ys)
        pltpu.sync_copy(z_hbm.at[pl.ds(base, CHUNK)], zs)

        def level_body(l, _):
            res = plsc.load_gather(res_v, [jnp.full((LANES,), l, jnp.int32)])
            lofs = l * T

            def grp1(j, _):
                x = xs[pl.ds(j * LANES, LANES)]
                y = ys[pl.ds(j * LANES, LANES)]
                z = zs[pl.ds(j * LANES, LANES)]
                sx = x * res
                sy = y * res
                sz = z * res
                fx = sx.astype(jnp.int32)
                fy = sy.astype(jnp.int32)
                fz = sz.astype(jnp.int32)
                frx = sx - fx.astype(jnp.float32)
                fry = sy - fy.astype(jnp.float32)
                frz = sz - fz.astype(jnp.float32)
                a = (fx, fx + 1)
                b = (fy * P1, fy * P1 + P1)
                c = (fz * P2, fz * P2 + P2)
                wx = (1.0 - frx, frx)
                wy = (1.0 - fry, fry)
                wz = (1.0 - frz, frz)
                for corner in range(8):
                    cx, cy, cz = corner & 1, (corner >> 1) & 1, (corner >> 2) & 1
                    h = ((a[cx] ^ b[cy] ^ c[cz]) & HASH_MASK) + lofs
                    fa = h + h
                    idxA[pl.ds(j * 8 * LANES + corner * LANES, LANES)] = fa
                    idxB[pl.ds(j * 8 * LANES + corner * LANES, LANES)] = fa + 1
                    w2d[j, pl.ds(corner * LANES, LANES)] = (wx[cx] * wy[cy]) * wz[cz]
                return 0

            lax.fori_loop(0, GROUPS, grp1, 0)
            cpA = pltpu.async_copy(tbl_hbm.at[idxA], rows0, semA)
            cpB = pltpu.async_copy(tbl_hbm.at[idxB], rows1, semB)
            cpA.wait()
            cpB.wait()

            def grp2(j, _):
                f0 = jnp.zeros((LANES,), jnp.float32)
                f1 = jnp.zeros((LANES,), jnp.float32)
                for corner in range(8):
                    pos = j * 8 * LANES + corner * LANES
                    wv = w2d[j, pl.ds(corner * LANES, LANES)]
                    f0 = f0 + wv * rows0[pl.ds(pos, LANES)]
                    f1 = f1 + wv * rows1[pl.ds(pos, LANES)]
                ebase = (j * LANES) * ENC_DIM + 2 * l
                prow = lane * ENC_DIM + ebase
                plsc.store_scatter(enc_buf, [prow], f0)
                plsc.store_scatter(enc_buf, [prow + 1], f1)
                return 0

            lax.fori_loop(0, GROUPS, grp2, 0)
            return 0

        lax.fori_loop(0, NUM_LEVELS, level_body, 0)
        pltpu.sync_copy(enc_buf, enc_hbm.at[pl.ds(base * ENC_DIM, CHUNK * ENC_DIM)])
        return 0

    lax.fori_loop(0, N_CHUNKS, chunk_body, 0)


@jax.jit
def _sc_encode(x, y, z, tbl, res_arr):
    mesh = plsc.VectorSubcoreMesh(core_axis_name="c", subcore_axis_name="s")
    f = pl.kernel(
        _sc_encode_body,
        out_type=jax.ShapeDtypeStruct((N_POINTS * ENC_DIM,), jnp.float32),
        mesh=mesh,
        compiler_params=pltpu.CompilerParams(
            needs_layout_passes=False, use_tc_tiling_on_sc=False),
        scratch_types=[
            pltpu.VMEM((CHUNK,), jnp.float32),
            pltpu.VMEM((CHUNK,), jnp.float32),
            pltpu.VMEM((CHUNK,), jnp.float32),
            pltpu.VMEM((GROUPS * 8 * LANES,), jnp.int32),
            pltpu.VMEM((GROUPS * 8 * LANES,), jnp.int32),
            pltpu.VMEM((GROUPS, 8 * LANES), jnp.float32),
            pltpu.VMEM((GROUPS * 8 * LANES,), jnp.float32),
            pltpu.VMEM((GROUPS * 8 * LANES,), jnp.float32),
            pltpu.VMEM((CHUNK * ENC_DIM,), jnp.float32),
            pltpu.VMEM((NUM_LEVELS,), jnp.float32),
            pltpu.SemaphoreType.DMA,
            pltpu.SemaphoreType.DMA,
        ],
    )
    return f(x, y, z, tbl, res_arr)


def _mlp_body(enc_ref, w1_ref, b1_ref, w2_ref, b2_ref, out_ref):
    h = jnp.dot(enc_ref[...], w1_ref[...], preferred_element_type=jnp.float32)
    h = jnp.maximum(h + b1_ref[...], 0.0)
    out_ref[...] = jnp.dot(h, w2_ref[...], preferred_element_type=jnp.float32) + b2_ref[...]


@jax.jit
def _mlp(enc, W1, b1, W2, b2):
    BN = 4096
    grid = (N_POINTS // BN,)
    return pl.pallas_call(
        _mlp_body,
        grid=grid,
        in_specs=[
            pl.BlockSpec((BN, ENC_DIM), lambda i: (i, 0)),
            pl.BlockSpec((ENC_DIM, 64), lambda i: (0, 0)),
            pl.BlockSpec((1, 64), lambda i: (0, 0)),
            pl.BlockSpec((64, NUM_FEATS), lambda i: (0, 0)),
            pl.BlockSpec((1, NUM_FEATS), lambda i: (0, 0)),
        ],
        out_specs=pl.BlockSpec((BN, NUM_FEATS), lambda i: (i, 0)),
        out_shape=jax.ShapeDtypeStruct((N_POINTS, NUM_FEATS), jnp.float32),
    )(enc, W1, b1, W2, b2)


def kernel(coords, tables, W1, b1, W2, b2):
    x = coords[:, 0]
    y = coords[:, 1]
    z = coords[:, 2]
    tbl = tables.reshape(NUM_LEVELS * T * NUM_FEATS)
    res_arr = jnp.asarray(RES_F32)
    enc = _sc_encode(x, y, z, tbl, res_arr).reshape(N_POINTS, ENC_DIM)
    return _mlp(enc, W1, b1.reshape(1, 64), W2, b2.reshape(1, NUM_FEATS))


# speedup vs baseline: 9.5536x; 8.4026x over previous
import jax
import jax.numpy as jnp
import numpy as np
from jax import lax
from jax.experimental import pallas as pl
from jax.experimental.pallas import tpu as pltpu
from jax.experimental.pallas import tpu_sc as plsc

NUM_LEVELS = 16
NUM_FEATS = 2
LOG2_T = 19
T = 2 ** LOG2_T
BASE_RES = 16.0
DESIRED_RES = 2048.0
N_POINTS = 262144

P1 = np.int32(np.int64(2654435761) - 2 ** 32)
P2 = np.int32(805459861)
HASH_MASK = T - 1

NC = 2
NS = 16
LANES = 16
NW = NC * NS
PTS_PER_WORKER = N_POINTS // NW
CHUNK = 512
N_CHUNKS = PTS_PER_WORKER // CHUNK
GROUPS = CHUNK // LANES
ENC_DIM = NUM_LEVELS * NUM_FEATS
NIDX = 8 * CHUNK
TBL_SIZE = NUM_LEVELS * T * NUM_FEATS

_growth = np.exp((np.log(DESIRED_RES) - np.log(BASE_RES)) / (NUM_LEVELS - 1))
RES_F32 = [np.float32(BASE_RES * _growth ** l) for l in range(NUM_LEVELS)]


PRE_CHUNK = 16384
PRE_PER_WORKER = TBL_SIZE // NW
PRE_ROUNDS = PRE_PER_WORKER // PRE_CHUNK


def _sc_pack_body(tbl_hbm, out_hbm, tin0, tin1, tout0, tout1,
                  semi0, semi1, semo0, semo1):
    wid = lax.axis_index("s") * NC + lax.axis_index("c")
    lane = lax.iota(jnp.int32, LANES)
    lane2 = lane + lane
    tin_b = (tin0, tin1)
    tout_b = (tout0, tout1)
    semi_b = (semi0, semi1)
    semo_b = (semo0, semo1)
    wbase = wid * PRE_PER_WORKER

    def fire_in(r):
        return pltpu.async_copy(
            tbl_hbm.at[pl.ds(wbase + r * PRE_CHUNK, PRE_CHUNK)],
            tin_b[r % 2], semi_b[r % 2])

    def fire_out(r):
        return pltpu.async_copy(
            tout_b[r % 2],
            out_hbm.at[pl.ds(wbase + r * PRE_CHUNK, PRE_CHUNK)],
            semo_b[r % 2])

    cin = {0: fire_in(0)}
    cout = {}
    for r in range(PRE_ROUNDS):
        tin = tin_b[r % 2]
        tout = tout_b[r % 2]
        if r + 1 < PRE_ROUNDS:
            cin[r + 1] = fire_in(r + 1)
        cin[r].wait()
        if r >= 2:
            cout[r - 2].wait()

        @plsc.parallel_loop(0, PRE_CHUNK // LANES, unroll=2)
        def grp(i):
            v = tin[pl.ds(i * LANES, LANES)]
            obase = ((i >> 4) << 8) + ((i & 7) << 5) + ((i >> 3) & 1)
            plsc.store_scatter(tout, [obase + lane2], v)

        cout[r] = fire_out(r)
    cout[PRE_ROUNDS - 2].wait()
    cout[PRE_ROUNDS - 1].wait()


@jax.jit
def _sc_pack(tbl):
    mesh = plsc.VectorSubcoreMesh(core_axis_name="c", subcore_axis_name="s")
    f = pl.kernel(
        _sc_pack_body,
        out_type=jax.ShapeDtypeStruct((TBL_SIZE,), jnp.float32),
        mesh=mesh,
        compiler_params=pltpu.CompilerParams(
            needs_layout_passes=False, use_tc_tiling_on_sc=False),
        scratch_types=[
            pltpu.VMEM((PRE_CHUNK,), jnp.float32),
            pltpu.VMEM((PRE_CHUNK,), jnp.float32),
            pltpu.VMEM((PRE_CHUNK,), jnp.float32),
            pltpu.VMEM((PRE_CHUNK,), jnp.float32),
            pltpu.SemaphoreType.DMA,
            pltpu.SemaphoreType.DMA,
            pltpu.SemaphoreType.DMA,
            pltpu.SemaphoreType.DMA,
        ],
    )
    return f(tbl)


def _sc_encode_body(x_hbm, y_hbm, z_hbm, tbl_hbm, enc_hbm,
                    xs, ys, zs, idx0, idx1, rem0, rem1, w0, w1, r0, r1,
                    enc_buf, sem0, sem1, semx0, semx1, semy0, semy1,
                    semz0, semz1):
    wid = lax.axis_index("s") * NC + lax.axis_index("c")
    lane = lax.iota(jnp.int32, LANES)
    idx_b = (idx0, idx1)
    rem_b = (rem0, rem1)
    w_b = (w0, w1)
    rows_b = (r0, r1)
    sems_b = ((sem0, semx0, semy0, semz0), (sem1, semx1, semy1, semz1))
    NSPLIT = 4
    PART = NIDX // NSPLIT

    def pass1(l, p):
        res = RES_F32[l]
        lofs4 = l * (T // 4)
        idx_ref = idx_b[p]
        rem_ref = rem_b[p]
        w_ref = w_b[p]

        @plsc.parallel_loop(0, GROUPS, unroll=2)
        def grp1(j):
            x = xs[pl.ds(j * LANES, LANES)]
            y = ys[pl.ds(j * LANES, LANES)]
            z = zs[pl.ds(j * LANES, LANES)]
            sx = x * res
            sy = y * res
            sz = z * res
            fx = sx.astype(jnp.int32)
            fy = sy.astype(jnp.int32)
            fz = sz.astype(jnp.int32)
            frx = sx - fx.astype(jnp.float32)
            fry = sy - fy.astype(jnp.float32)
            frz = sz - fz.astype(jnp.float32)
            a = (fx, fx + 1)
            b = (fy * P1, fy * P1 + P1)
            c = (fz * P2, fz * P2 + P2)
            wx = (1.0 - frx, frx)
            wy = (1.0 - fry, fry)
            wz = (1.0 - frz, frz)
            for corner in range(8):
                cx, cy, cz = corner & 1, (corner >> 1) & 1, (corner >> 2) & 1
                h = (a[cx] ^ b[cy] ^ c[cz]) & HASH_MASK
                pos = j * 8 * LANES + corner * LANES
                idx_ref[pl.ds(pos, LANES)] = (h >> 2) + lofs4
                rem_ref[pl.ds(pos, LANES)] = (h & 3) + (h & 3)
                w_ref[pl.ds(pos, LANES)] = (wx[cx] * wy[cy]) * wz[cz]

    def fire(p):
        for s in range(NSPLIT):
            pltpu.async_copy(tbl_hbm.at[idx_b[p].at[pl.ds(s * PART, PART)]],
                             rows_b[p].at[pl.ds(s * PART, PART)], sems_b[p][s])

    def pass2(l, p):
        w_ref = w_b[p]
        rem_ref = rem_b[p]
        rows = rows_b[p]

        @plsc.parallel_loop(0, GROUPS, unroll=2)
        def grp2(j):
            f0 = jnp.zeros((LANES,), jnp.float32)
            f1 = jnp.zeros((LANES,), jnp.float32)
            for corner in range(8):
                pos = j * 8 * LANES + corner * LANES
                wv = w_ref[pl.ds(pos, LANES)]
                remv = rem_ref[pl.ds(pos, LANES)]
                posv = lane + pos
                f0 = f0 + wv * plsc.load_gather(rows, [posv, remv])
                f1 = f1 + wv * plsc.load_gather(rows, [posv, remv + 1])
            prow = lane * ENC_DIM + (j * LANES * ENC_DIM + 2 * l)
            plsc.store_scatter(enc_buf, [prow], f0)
            plsc.store_scatter(enc_buf, [prow + 1], f1)

    def load_xyz(ci):
        base = wid * PTS_PER_WORKER + ci * CHUNK
        pltpu.sync_copy(x_hbm.at[pl.ds(base, CHUNK)], xs)
        pltpu.sync_copy(y_hbm.at[pl.ds(base, CHUNK)], ys)
        pltpu.sync_copy(z_hbm.at[pl.ds(base, CHUNK)], zs)

    def wait_gather(p):
        for s in range(NSPLIT):
            pltpu.make_async_copy(
                tbl_hbm.at[idx_b[p].at[pl.ds(s * PART, PART)]],
                rows_b[p].at[pl.ds(s * PART, PART)], sems_b[p][s]).wait()

    load_xyz(0)
    pass1(0, 0)
    fire(0)

    def chunk_body(ci, _):
        base = wid * PTS_PER_WORKER + ci * CHUNK
        for l in range(NUM_LEVELS):
            p = l % 2
            q = (l + 1) % 2
            if l + 1 < NUM_LEVELS:
                pass1(l + 1, q)
            else:
                cnext = jnp.minimum(ci + 1, N_CHUNKS - 1)
                load_xyz(cnext)
                pass1(0, q)
            wait_gather(p)
            fire(q)
            pass2(l, p)
        pltpu.sync_copy(enc_buf, enc_hbm.at[pl.ds(base * ENC_DIM, CHUNK * ENC_DIM)])
        return 0

    lax.fori_loop(0, N_CHUNKS, chunk_body, 0)
    wait_gather(0)


@jax.jit
def _sc_encode(x, y, z, tbl):
    mesh = plsc.VectorSubcoreMesh(core_axis_name="c", subcore_axis_name="s")
    f = pl.kernel(
        _sc_encode_body,
        out_type=jax.ShapeDtypeStruct((N_POINTS * ENC_DIM,), jnp.float32),
        mesh=mesh,
        compiler_params=pltpu.CompilerParams(
            needs_layout_passes=False, use_tc_tiling_on_sc=False),
        scratch_types=[
            pltpu.VMEM((CHUNK,), jnp.float32),
            pltpu.VMEM((CHUNK,), jnp.float32),
            pltpu.VMEM((CHUNK,), jnp.float32),
            pltpu.VMEM((NIDX,), jnp.int32),
            pltpu.VMEM((NIDX,), jnp.int32),
            pltpu.VMEM((NIDX,), jnp.int32),
            pltpu.VMEM((NIDX,), jnp.int32),
            pltpu.VMEM((NIDX,), jnp.float32),
            pltpu.VMEM((NIDX,), jnp.float32),
            pltpu.VMEM((NIDX, 8), jnp.float32),
            pltpu.VMEM((NIDX, 8), jnp.float32),
            pltpu.VMEM((CHUNK * ENC_DIM,), jnp.float32),
            pltpu.SemaphoreType.DMA,
            pltpu.SemaphoreType.DMA,
            pltpu.SemaphoreType.DMA,
            pltpu.SemaphoreType.DMA,
            pltpu.SemaphoreType.DMA,
            pltpu.SemaphoreType.DMA,
            pltpu.SemaphoreType.DMA,
            pltpu.SemaphoreType.DMA,
        ],
    )
    return f(x, y, z, tbl)


PTS_PER_ROW = 128 // ENC_DIM
MLP_ROWS = N_POINTS // PTS_PER_ROW


def _mlp_body(enc_ref, w1_ref, b1_ref, w2_ref, b2_ref, out_ref):
    h = jnp.dot(enc_ref[...], w1_ref[...], preferred_element_type=jnp.float32)
    h = jnp.maximum(h + b1_ref[...], 0.0)
    out_ref[...] = jnp.dot(h, w2_ref[...], preferred_element_type=jnp.float32) + b2_ref[...]


@jax.jit
def _mlp(enc4, W1b, b1b, W2b, b2b):
    BN = 2048
    grid = (MLP_ROWS // BN,)
    return pl.pallas_call(
        _mlp_body,
        grid=grid,
        in_specs=[
            pl.BlockSpec((BN, 128), lambda i: (i, 0)),
            pl.BlockSpec((128, 256), lambda i: (0, 0)),
            pl.BlockSpec((1, 256), lambda i: (0, 0)),
            pl.BlockSpec((256, PTS_PER_ROW * NUM_FEATS), lambda i: (0, 0)),
            pl.BlockSpec((1, PTS_PER_ROW * NUM_FEATS), lambda i: (0, 0)),
        ],
        out_specs=pl.BlockSpec((BN, PTS_PER_ROW * NUM_FEATS), lambda i: (i, 0)),
        out_shape=jax.ShapeDtypeStruct((MLP_ROWS, PTS_PER_ROW * NUM_FEATS), jnp.float32),
    )(enc4, W1b, b1b, W2b, b2b)


def _block_diag4(W):
    k, m = W.shape
    out = jnp.zeros((PTS_PER_ROW * k, PTS_PER_ROW * m), W.dtype)
    for i in range(PTS_PER_ROW):
        out = out.at[i * k:(i + 1) * k, i * m:(i + 1) * m].set(W)
    return out


def kernel(coords, tables, W1, b1, W2, b2):
    x = coords[:, 0]
    y = coords[:, 1]
    z = coords[:, 2]
    tbl = tables.reshape(NUM_LEVELS, T // 128, 128, NUM_FEATS)
    tbl = tbl.transpose(0, 1, 3, 2).reshape(TBL_SIZE)
    tbl_pairs = _sc_pack(tbl).reshape(TBL_SIZE // 8, 8)
    enc4 = _sc_encode(x, y, z, tbl_pairs).reshape(MLP_ROWS, 128)
    W1b = _block_diag4(W1)
    b1b = jnp.tile(b1, PTS_PER_ROW).reshape(1, 256)
    W2b = _block_diag4(W2)
    b2b = jnp.tile(b2, PTS_PER_ROW).reshape(1, PTS_PER_ROW * NUM_FEATS)
    out4 = _mlp(enc4, W1b, b1b, W2b, b2b)
    return out4.reshape(N_POINTS, NUM_FEATS)
